# baseline (speedup 1.0000x reference)
import functools

import jax
import jax.numpy as jnp
from jax import lax
from jax.experimental import pallas as pl
from jax.experimental.pallas import tpu as pltpu
from jax.experimental.pallas import tpu_sc as plsc

B, D = 4096, 12288
TBL_PAD = 1024

_NC, _NS = 2, 16
_NW = _NC * _NS
_CHUNK = B // _NW
_LANES = 16


def _make_sc_gather():
    mesh = plsc.VectorSubcoreMesh(core_axis_name="c", subcore_axis_name="s")

    @functools.partial(
        pl.kernel,
        mesh=mesh,
        out_type=(
            jax.ShapeDtypeStruct((B,), jnp.float32),
            jax.ShapeDtypeStruct((B,), jnp.float32),
        ),
        scratch_types=[
            pltpu.VMEM((_CHUNK,), jnp.int32),
            pltpu.VMEM((_CHUNK,), jnp.float32),
            pltpu.VMEM((_CHUNK,), jnp.float32),
            pltpu.SemaphoreType.DMA,
            pltpu.SemaphoreType.DMA,
        ],
    )
    def sc_gather(t_hbm, sa_hbm, sb_hbm, oa_hbm, ob_hbm,
                  idx_v, oa_v, ob_v, sem_a, sem_b):
        wid = lax.axis_index("s") * _NC + lax.axis_index("c")
        base = wid * _CHUNK
        pltpu.sync_copy(t_hbm.at[pl.ds(base, _CHUNK)], idx_v)
        cp_a = pltpu.async_copy(sa_hbm.at[idx_v], oa_v, sem_a)
        cp_b = pltpu.async_copy(sb_hbm.at[idx_v], ob_v, sem_b)
        cp_a.wait()
        cp_b.wait()
        pltpu.sync_copy(oa_v, oa_hbm.at[pl.ds(base, _CHUNK)])
        pltpu.sync_copy(ob_v, ob_hbm.at[pl.ds(base, _CHUNK)])

    return sc_gather


_BR = 256
_BC = 6144

R_TC = 3072
R_SC = B - R_TC
_RPT = R_SC // _NW
_NBUF = 2


def _make_sc_dense():
    mesh = plsc.VectorSubcoreMesh(core_axis_name="c", subcore_axis_name="s")

    @functools.partial(
        pl.kernel,
        mesh=mesh,
        out_type=jax.ShapeDtypeStruct((R_SC, D), jnp.float32),
        scratch_types=[
            pltpu.VMEM((_NBUF, D), jnp.float32),
            pltpu.VMEM((_NBUF, D), jnp.float32),
            pltpu.VMEM((_NBUF, D), jnp.float32),
            pltpu.VMEM((_RPT * _LANES,), jnp.float32),
            pltpu.VMEM((_RPT * _LANES,), jnp.float32),
            pltpu.SemaphoreType.DMA,
            pltpu.SemaphoreType.DMA,
            pltpu.SemaphoreType.DMA,
            pltpu.SemaphoreType.DMA,
            pltpu.SemaphoreType.DMA,
            pltpu.SemaphoreType.DMA,
            pltpu.SemaphoreType.DMA,
        ],
    )
    def sc_dense(x_hbm, n_hbm, sa_hbm, sb_hbm, o_hbm,
                 xb, nb, ob, ab, bb,
                 sx0, sx1, sn0, sn1, so0, so1, sg):
        wid = lax.axis_index("s") * _NC + lax.axis_index("c")
        row0 = wid * _RPT
        sems_x, sems_n, sems_o = (sx0, sx1), (sn0, sn1), (so0, so1)

        gcps = []
        for r in range(_RPT):
            idx = jnp.full((_LANES,), R_TC + row0 + r, jnp.int32)
            gcps.append(pltpu.async_copy(
                sa_hbm.at[idx], ab.at[pl.ds(r * _LANES, _LANES)], sg))
            gcps.append(pltpu.async_copy(
                sb_hbm.at[idx], bb.at[pl.ds(r * _LANES, _LANES)], sg))
        for cp in gcps:
            cp.wait()

        cx = [None] * _NBUF
        cn = [None] * _NBUF
        co = [None] * _NBUF

        def start_load(k):
            buf = k % _NBUF
            row = R_TC + row0 + k
            cx[buf] = pltpu.async_copy(x_hbm.at[row], xb.at[buf], sems_x[buf])
            cn[buf] = pltpu.async_copy(n_hbm.at[row], nb.at[buf], sems_n[buf])

        _U = 8

        start_load(0)
        for k in range(_RPT):
            buf = k % _NBUF
            if k + 1 < _RPT:
                start_load(k + 1)
            cx[buf].wait()
            cn[buf].wait()
            if co[buf] is not None:
                co[buf].wait()
            av = ab[pl.ds(k * _LANES, _LANES)]
            bv = bb[pl.ds(k * _LANES, _LANES)]

            def body(i, _, buf=buf, av=av, bv=bv):
                base = i * (_U * _LANES)
                for u in range(_U):
                    sl = pl.ds(base + u * _LANES, _LANES)
                    ob[buf, sl] = jnp.clip(
                        av * xb[buf, sl] + bv * nb[buf, sl], -1.0, 1.0)
                return 0

            lax.fori_loop(0, D // (_U * _LANES), body, 0)
            co[buf] = pltpu.async_copy(
                ob.at[buf], o_hbm.at[row0 + k], sems_o[buf])
        for buf in range(_NBUF):
            if co[buf] is not None:
                co[buf].wait()

    return sc_dense


def _tc_body(sa_ref, sb_ref, x_ref, n_ref, o_ref):
    o_ref[...] = jnp.clip(
        sa_ref[...] * x_ref[...] + sb_ref[...] * n_ref[...], -1.0, 1.0)


def kernel(x_0, t, noise, sqrt_alpha, sqrt_1m_alpha):
    scale_a, scale_b = _make_sc_gather()(t, sqrt_alpha, sqrt_1m_alpha)
    sc_out = _make_sc_dense()(x_0, noise, scale_a, scale_b)
    tc_out = pl.pallas_call(
        _tc_body,
        grid=(R_TC // _BR, D // _BC),
        in_specs=[
            pl.BlockSpec((_BR, 1), lambda i, j: (i, 0)),
            pl.BlockSpec((_BR, 1), lambda i, j: (i, 0)),
            pl.BlockSpec((_BR, _BC), lambda i, j: (i, j)),
            pl.BlockSpec((_BR, _BC), lambda i, j: (i, j)),
        ],
        out_specs=pl.BlockSpec((_BR, _BC), lambda i, j: (i, j)),
        out_shape=jax.ShapeDtypeStruct((R_TC, D), jnp.float32),
    )(scale_a.reshape(B, 1), scale_b.reshape(B, 1), x_0, noise)
    return jnp.concatenate([tc_out, sc_out], axis=0), noise


# speedup vs baseline: 1.1211x; 1.1211x over previous
import functools

import jax
import jax.numpy as jnp
from jax import lax
from jax.experimental import pallas as pl
from jax.experimental.pallas import tpu as pltpu
from jax.experimental.pallas import tpu_sc as plsc

B, D = 4096, 12288
TBL_PAD = 1024

_NC, _NS = 2, 16
_NW = _NC * _NS
_CHUNK = B // _NW
_LANES = 16


def _make_sc_gather():
    mesh = plsc.VectorSubcoreMesh(core_axis_name="c", subcore_axis_name="s")

    @functools.partial(
        pl.kernel,
        mesh=mesh,
        out_type=(
            jax.ShapeDtypeStruct((B,), jnp.float32),
            jax.ShapeDtypeStruct((B,), jnp.float32),
        ),
        scratch_types=[
            pltpu.VMEM((_CHUNK,), jnp.int32),
            pltpu.VMEM((_CHUNK,), jnp.float32),
            pltpu.VMEM((_CHUNK,), jnp.float32),
            pltpu.SemaphoreType.DMA,
            pltpu.SemaphoreType.DMA,
        ],
    )
    def sc_gather(t_hbm, sa_hbm, sb_hbm, oa_hbm, ob_hbm,
                  idx_v, oa_v, ob_v, sem_a, sem_b):
        wid = lax.axis_index("s") * _NC + lax.axis_index("c")
        base = wid * _CHUNK
        pltpu.sync_copy(t_hbm.at[pl.ds(base, _CHUNK)], idx_v)
        cp_a = pltpu.async_copy(sa_hbm.at[idx_v], oa_v, sem_a)
        cp_b = pltpu.async_copy(sb_hbm.at[idx_v], ob_v, sem_b)
        cp_a.wait()
        cp_b.wait()
        pltpu.sync_copy(oa_v, oa_hbm.at[pl.ds(base, _CHUNK)])
        pltpu.sync_copy(ob_v, ob_hbm.at[pl.ds(base, _CHUNK)])

    return sc_gather


_BR = 256
_BC = 4096

R_TC = 3072
R_SC = B - R_TC
_RPT = R_SC // _NW
_NBUF = 2


def _make_sc_dense():
    mesh = plsc.VectorSubcoreMesh(core_axis_name="c", subcore_axis_name="s")

    @functools.partial(
        pl.kernel,
        mesh=mesh,
        out_type=jax.ShapeDtypeStruct((R_SC, D), jnp.float32),
        scratch_types=[
            pltpu.VMEM((_NBUF, D), jnp.float32),
            pltpu.VMEM((_NBUF, D), jnp.float32),
            pltpu.VMEM((_NBUF, D), jnp.float32),
            pltpu.VMEM((_RPT * _LANES,), jnp.float32),
            pltpu.VMEM((_RPT * _LANES,), jnp.float32),
            pltpu.SemaphoreType.DMA,
            pltpu.SemaphoreType.DMA,
            pltpu.SemaphoreType.DMA,
            pltpu.SemaphoreType.DMA,
            pltpu.SemaphoreType.DMA,
            pltpu.SemaphoreType.DMA,
            pltpu.SemaphoreType.DMA,
        ],
    )
    def sc_dense(x_hbm, n_hbm, sa_hbm, sb_hbm, o_hbm,
                 xb, nb, ob, ab, bb,
                 sx0, sx1, sn0, sn1, so0, so1, sg):
        wid = lax.axis_index("s") * _NC + lax.axis_index("c")
        row0 = wid * _RPT
        sems_x, sems_n, sems_o = (sx0, sx1), (sn0, sn1), (so0, so1)

        gcps = []
        for r in range(_RPT):
            idx = jnp.full((_LANES,), R_TC + row0 + r, jnp.int32)
            gcps.append(pltpu.async_copy(
                sa_hbm.at[idx], ab.at[pl.ds(r * _LANES, _LANES)], sg))
            gcps.append(pltpu.async_copy(
                sb_hbm.at[idx], bb.at[pl.ds(r * _LANES, _LANES)], sg))
        for cp in gcps:
            cp.wait()

        cx = [None] * _NBUF
        cn = [None] * _NBUF
        co = [None] * _NBUF

        def start_load(k):
            buf = k % _NBUF
            row = R_TC + row0 + k
            cx[buf] = pltpu.async_copy(x_hbm.at[row], xb.at[buf], sems_x[buf])
            cn[buf] = pltpu.async_copy(n_hbm.at[row], nb.at[buf], sems_n[buf])

        _U = 8

        start_load(0)
        for k in range(_RPT):
            buf = k % _NBUF
            if k + 1 < _RPT:
                start_load(k + 1)
            cx[buf].wait()
            cn[buf].wait()
            if co[buf] is not None:
                co[buf].wait()
            av = ab[pl.ds(k * _LANES, _LANES)]
            bv = bb[pl.ds(k * _LANES, _LANES)]

            def body(i, _, buf=buf, av=av, bv=bv):
                base = i * (_U * _LANES)
                for u in range(_U):
                    sl = pl.ds(base + u * _LANES, _LANES)
                    ob[buf, sl] = jnp.clip(
                        av * xb[buf, sl] + bv * nb[buf, sl], -1.0, 1.0)
                return 0

            lax.fori_loop(0, D // (_U * _LANES), body, 0)
            co[buf] = pltpu.async_copy(
                ob.at[buf], o_hbm.at[row0 + k], sems_o[buf])
        for buf in range(_NBUF):
            if co[buf] is not None:
                co[buf].wait()

    return sc_dense


def _tc_body(sa_ref, sb_ref, x_ref, n_ref, sc_ref, o_ref):
    i = pl.program_id(1)

    @pl.when(i < R_TC // _BR)
    def _compute():
        o_ref[...] = jnp.clip(
            sa_ref[...] * x_ref[...] + sb_ref[...] * n_ref[...], -1.0, 1.0)

    @pl.when(i >= R_TC // _BR)
    def _passthrough():
        o_ref[...] = sc_ref[...]


def kernel(x_0, t, noise, sqrt_alpha, sqrt_1m_alpha):
    scale_a, scale_b = _make_sc_gather()(t, sqrt_alpha, sqrt_1m_alpha)
    sc_out = _make_sc_dense()(x_0, noise, scale_a, scale_b)
    n_tc = R_TC // _BR
    out = pl.pallas_call(
        _tc_body,
        grid=(D // _BC, B // _BR),
        in_specs=[
            pl.BlockSpec((_BR, 1), lambda j, i: (i, 0)),
            pl.BlockSpec((_BR, 1), lambda j, i: (i, 0)),
            pl.BlockSpec((_BR, _BC),
                         lambda j, i: (jnp.minimum(i, n_tc - 1), j)),
            pl.BlockSpec((_BR, _BC),
                         lambda j, i: (jnp.minimum(i, n_tc - 1), j)),
            pl.BlockSpec((_BR, _BC),
                         lambda j, i: (jnp.maximum(i - n_tc, 0), j)),
        ],
        out_specs=pl.BlockSpec((_BR, _BC), lambda j, i: (i, j)),
        out_shape=jax.ShapeDtypeStruct((B, D), jnp.float32),
    )(scale_a.reshape(B, 1), scale_b.reshape(B, 1), x_0, noise, sc_out)
    return out, noise


# speedup vs baseline: 1.1739x; 1.0470x over previous
import functools

import jax
import jax.numpy as jnp
from jax import lax
from jax.experimental import pallas as pl
from jax.experimental.pallas import tpu as pltpu
from jax.experimental.pallas import tpu_sc as plsc

B, D = 4096, 12288
TBL_PAD = 1024

_NC, _NS = 2, 16
_NW = _NC * _NS
_CHUNK = B // _NW
_LANES = 16


def _make_sc_gather():
    mesh = plsc.VectorSubcoreMesh(core_axis_name="c", subcore_axis_name="s")

    @functools.partial(
        pl.kernel,
        mesh=mesh,
        out_type=(
            jax.ShapeDtypeStruct((B,), jnp.float32),
            jax.ShapeDtypeStruct((B,), jnp.float32),
        ),
        scratch_types=[
            pltpu.VMEM((_CHUNK,), jnp.int32),
            pltpu.VMEM((_CHUNK,), jnp.float32),
            pltpu.VMEM((_CHUNK,), jnp.float32),
            pltpu.SemaphoreType.DMA,
            pltpu.SemaphoreType.DMA,
        ],
    )
    def sc_gather(t_hbm, sa_hbm, sb_hbm, oa_hbm, ob_hbm,
                  idx_v, oa_v, ob_v, sem_a, sem_b):
        wid = lax.axis_index("s") * _NC + lax.axis_index("c")
        base = wid * _CHUNK
        pltpu.sync_copy(t_hbm.at[pl.ds(base, _CHUNK)], idx_v)
        cp_a = pltpu.async_copy(sa_hbm.at[idx_v], oa_v, sem_a)
        cp_b = pltpu.async_copy(sb_hbm.at[idx_v], ob_v, sem_b)
        cp_a.wait()
        cp_b.wait()
        pltpu.sync_copy(oa_v, oa_hbm.at[pl.ds(base, _CHUNK)])
        pltpu.sync_copy(ob_v, ob_hbm.at[pl.ds(base, _CHUNK)])

    return sc_gather


_BR = 256
_BC = 4096

R_TC = 2816
R_SC = B - R_TC
_RPT = R_SC // _NW
_NBUF = 2


def _make_sc_dense():
    mesh = plsc.VectorSubcoreMesh(core_axis_name="c", subcore_axis_name="s")

    @functools.partial(
        pl.kernel,
        mesh=mesh,
        out_type=jax.ShapeDtypeStruct((R_SC, D), jnp.float32),
        scratch_types=[
            pltpu.VMEM((_NBUF, D), jnp.float32),
            pltpu.VMEM((_NBUF, D), jnp.float32),
            pltpu.VMEM((_NBUF, D), jnp.float32),
            pltpu.VMEM((_RPT * _LANES,), jnp.int32),
            pltpu.VMEM((_RPT * _LANES,), jnp.float32),
            pltpu.VMEM((_RPT * _LANES,), jnp.float32),
            pltpu.SemaphoreType.DMA,
            pltpu.SemaphoreType.DMA,
            pltpu.SemaphoreType.DMA,
            pltpu.SemaphoreType.DMA,
            pltpu.SemaphoreType.DMA,
            pltpu.SemaphoreType.DMA,
            pltpu.SemaphoreType.DMA,
        ],
    )
    def sc_dense(x_hbm, n_hbm, t_hbm, sa_hbm, sb_hbm, o_hbm,
                 xb, nb, ob, tb, ab, bb,
                 sx0, sx1, sn0, sn1, so0, so1, sg):
        wid = lax.axis_index("s") * _NC + lax.axis_index("c")
        row0 = wid * _RPT
        sems_x, sems_n, sems_o = (sx0, sx1), (sn0, sn1), (so0, so1)

        gcps = []
        for r in range(_RPT):
            idx = jnp.full((_LANES,), R_TC + row0 + r, jnp.int32)
            gcps.append(pltpu.async_copy(
                t_hbm.at[idx], tb.at[pl.ds(r * _LANES, _LANES)], sg))
        for cp in gcps:
            cp.wait()
        gcps = []
        for r in range(_RPT):
            tv = tb[pl.ds(r * _LANES, _LANES)]
            gcps.append(pltpu.async_copy(
                sa_hbm.at[tv], ab.at[pl.ds(r * _LANES, _LANES)], sg))
            gcps.append(pltpu.async_copy(
                sb_hbm.at[tv], bb.at[pl.ds(r * _LANES, _LANES)], sg))
        for cp in gcps:
            cp.wait()

        cx = [None] * _NBUF
        cn = [None] * _NBUF
        co = [None] * _NBUF

        def start_load(k):
            buf = k % _NBUF
            row = R_TC + row0 + k
            cx[buf] = pltpu.async_copy(x_hbm.at[row], xb.at[buf], sems_x[buf])
            cn[buf] = pltpu.async_copy(n_hbm.at[row], nb.at[buf], sems_n[buf])

        _U = 8

        start_load(0)
        for k in range(_RPT):
            buf = k % _NBUF
            if k + 1 < _RPT:
                start_load(k + 1)
            cx[buf].wait()
            cn[buf].wait()
            if co[buf] is not None:
                co[buf].wait()
            av = ab[pl.ds(k * _LANES, _LANES)]
            bv = bb[pl.ds(k * _LANES, _LANES)]

            def body(i, _, buf=buf, av=av, bv=bv):
                base = i * (_U * _LANES)
                for u in range(_U):
                    sl = pl.ds(base + u * _LANES, _LANES)
                    ob[buf, sl] = jnp.clip(
                        av * xb[buf, sl] + bv * nb[buf, sl], -1.0, 1.0)
                return 0

            lax.fori_loop(0, D // (_U * _LANES), body, 0)
            co[buf] = pltpu.async_copy(
                ob.at[buf], o_hbm.at[row0 + k], sems_o[buf])
        for buf in range(_NBUF):
            if co[buf] is not None:
                co[buf].wait()

    return sc_dense


def _tc_body(sa_ref, sb_ref, x_ref, n_ref, o_ref):
    o_ref[...] = jnp.clip(
        sa_ref[...] * x_ref[...] + sb_ref[...] * n_ref[...], -1.0, 1.0)


def _tc_copy_body(base_ref, sc_ref, o_ref):
    del base_ref
    o_ref[...] = sc_ref[...]


def kernel(x_0, t, noise, sqrt_alpha, sqrt_1m_alpha):
    scale_a, scale_b = _make_sc_gather()(t, sqrt_alpha, sqrt_1m_alpha)
    sc_out = _make_sc_dense()(x_0, noise, t, sqrt_alpha, sqrt_1m_alpha)
    head = pl.pallas_call(
        _tc_body,
        grid=(R_TC // _BR, D // _BC),
        in_specs=[
            pl.BlockSpec((_BR, 1), lambda i, j: (i, 0)),
            pl.BlockSpec((_BR, 1), lambda i, j: (i, 0)),
            pl.BlockSpec((_BR, _BC), lambda i, j: (i, j)),
            pl.BlockSpec((_BR, _BC), lambda i, j: (i, j)),
        ],
        out_specs=pl.BlockSpec((_BR, _BC), lambda i, j: (i, j)),
        out_shape=jax.ShapeDtypeStruct((B, D), jnp.float32),
    )(scale_a.reshape(B, 1), scale_b.reshape(B, 1), x_0, noise)
    out = pl.pallas_call(
        _tc_copy_body,
        grid=(R_SC // _BR,),
        in_specs=[
            pl.BlockSpec(memory_space=pl.ANY),
            pl.BlockSpec((_BR, D), lambda i: (i, 0)),
        ],
        out_specs=pl.BlockSpec((_BR, D), lambda i: (R_TC // _BR + i, 0)),
        out_shape=jax.ShapeDtypeStruct((B, D), jnp.float32),
        input_output_aliases={0: 0},
    )(head, sc_out)
    return out, noise


# speedup vs baseline: 1.7105x; 1.4572x over previous
import functools

import jax
import jax.numpy as jnp
from jax import lax
from jax.experimental import pallas as pl
from jax.experimental.pallas import tpu as pltpu
from jax.experimental.pallas import tpu_sc as plsc

B, D = 4096, 12288

_NC, _NS = 2, 16
_NW = _NC * _NS
_CHUNK = B // _NW
_LANES = 16


def _make_sc_gather():
    mesh = plsc.VectorSubcoreMesh(core_axis_name="c", subcore_axis_name="s")

    @functools.partial(
        pl.kernel,
        mesh=mesh,
        out_type=(
            jax.ShapeDtypeStruct((B,), jnp.float32),
            jax.ShapeDtypeStruct((B,), jnp.float32),
        ),
        scratch_types=[
            pltpu.VMEM((_CHUNK,), jnp.int32),
            pltpu.VMEM((_CHUNK,), jnp.float32),
            pltpu.VMEM((_CHUNK,), jnp.float32),
            pltpu.SemaphoreType.DMA,
            pltpu.SemaphoreType.DMA,
        ],
    )
    def sc_gather(t_hbm, sa_hbm, sb_hbm, oa_hbm, ob_hbm,
                  idx_v, oa_v, ob_v, sem_a, sem_b):
        wid = lax.axis_index("s") * _NC + lax.axis_index("c")
        base = wid * _CHUNK
        pltpu.sync_copy(t_hbm.at[pl.ds(base, _CHUNK)], idx_v)
        cp_a = pltpu.async_copy(sa_hbm.at[idx_v], oa_v, sem_a)
        cp_b = pltpu.async_copy(sb_hbm.at[idx_v], ob_v, sem_b)
        cp_a.wait()
        cp_b.wait()
        pltpu.sync_copy(oa_v, oa_hbm.at[pl.ds(base, _CHUNK)])
        pltpu.sync_copy(ob_v, ob_hbm.at[pl.ds(base, _CHUNK)])

    return sc_gather


_BR = 256
_BC = 4096


def _tc_body(sa_ref, sb_ref, x_ref, n_ref, o_ref, nc_ref):
    nv = n_ref[...]
    o_ref[...] = jnp.clip(
        sa_ref[...] * x_ref[...] + sb_ref[...] * nv, -1.0, 1.0)
    nc_ref[...] = nv


def kernel(x_0, t, noise, sqrt_alpha, sqrt_1m_alpha):
    scale_a, scale_b = _make_sc_gather()(t, sqrt_alpha, sqrt_1m_alpha)
    out, n_copy = pl.pallas_call(
        _tc_body,
        grid=(B // _BR, D // _BC),
        in_specs=[
            pl.BlockSpec((_BR, 1), lambda i, j: (i, 0)),
            pl.BlockSpec((_BR, 1), lambda i, j: (i, 0)),
            pl.BlockSpec((_BR, _BC), lambda i, j: (i, j)),
            pl.BlockSpec((_BR, _BC), lambda i, j: (i, j)),
        ],
        out_specs=[
            pl.BlockSpec((_BR, _BC), lambda i, j: (i, j)),
            pl.BlockSpec((_BR, _BC), lambda i, j: (i, j)),
        ],
        out_shape=[
            jax.ShapeDtypeStruct((B, D), jnp.float32),
            jax.ShapeDtypeStruct((B, D), jnp.float32),
        ],
    )(scale_a.reshape(B, 1), scale_b.reshape(B, 1), x_0, noise)
    return out, n_copy


# speedup vs baseline: 1.7538x; 1.0253x over previous
import functools

import jax
import jax.numpy as jnp
from jax import lax
from jax.experimental import pallas as pl
from jax.experimental.pallas import tpu as pltpu
from jax.experimental.pallas import tpu_sc as plsc

B, D = 4096, 12288

_NC, _NS = 2, 16
_NW = _NC * _NS
_CHUNK = B // _NW
_LANES = 16


def _make_sc_gather():
    mesh = plsc.VectorSubcoreMesh(core_axis_name="c", subcore_axis_name="s")

    @functools.partial(
        pl.kernel,
        mesh=mesh,
        out_type=(
            jax.ShapeDtypeStruct((B,), jnp.float32),
            jax.ShapeDtypeStruct((B,), jnp.float32),
        ),
        scratch_types=[
            pltpu.VMEM((_CHUNK,), jnp.int32),
            pltpu.VMEM((_CHUNK,), jnp.float32),
            pltpu.VMEM((_CHUNK,), jnp.float32),
            pltpu.SemaphoreType.DMA,
            pltpu.SemaphoreType.DMA,
        ],
    )
    def sc_gather(t_hbm, sa_hbm, sb_hbm, oa_hbm, ob_hbm,
                  idx_v, oa_v, ob_v, sem_a, sem_b):
        wid = lax.axis_index("s") * _NC + lax.axis_index("c")
        base = wid * _CHUNK
        pltpu.sync_copy(t_hbm.at[pl.ds(base, _CHUNK)], idx_v)
        cp_a = pltpu.async_copy(sa_hbm.at[idx_v], oa_v, sem_a)
        cp_b = pltpu.async_copy(sb_hbm.at[idx_v], ob_v, sem_b)
        cp_a.wait()
        cp_b.wait()
        pltpu.sync_copy(oa_v, oa_hbm.at[pl.ds(base, _CHUNK)])
        pltpu.sync_copy(ob_v, ob_hbm.at[pl.ds(base, _CHUNK)])

    return sc_gather


_BR = 256
_BC = 4096


def _tc_body(sa_ref, sb_ref, x_ref, n_ref, o_ref, nc_ref):
    nv = n_ref[...]
    sa = sa_ref[...].reshape(_BR, 1)
    sb = sb_ref[...].reshape(_BR, 1)
    o_ref[...] = jnp.clip(sa * x_ref[...] + sb * nv, -1.0, 1.0)
    nc_ref[...] = nv


def kernel(x_0, t, noise, sqrt_alpha, sqrt_1m_alpha):
    scale_a, scale_b = _make_sc_gather()(t, sqrt_alpha, sqrt_1m_alpha)
    out, n_copy = pl.pallas_call(
        _tc_body,
        grid=(B // _BR, D // _BC),
        in_specs=[
            pl.BlockSpec((_BR,), lambda i, j: (i,)),
            pl.BlockSpec((_BR,), lambda i, j: (i,)),
            pl.BlockSpec((_BR, _BC), lambda i, j: (i, j)),
            pl.BlockSpec((_BR, _BC), lambda i, j: (i, j)),
        ],
        out_specs=[
            pl.BlockSpec((_BR, _BC), lambda i, j: (i, j)),
            pl.BlockSpec((_BR, _BC), lambda i, j: (i, j)),
        ],
        out_shape=[
            jax.ShapeDtypeStruct((B, D), jnp.float32),
            jax.ShapeDtypeStruct((B, D), jnp.float32),
        ],
    )(scale_a, scale_b, x_0, noise)
    return out, n_copy
